# Initial kernel scaffold; baseline (speedup 1.0000x reference)
import jax
import jax.numpy as jnp
from jax.experimental import pallas as pl


def kernel(x, bases1, comp1, root1, bias1, bases2, comp2, root2, bias2, edge_index, edge_type):
    raise NotImplementedError("write your pallas kernel here")


# speedup vs baseline: 23.1577x; 23.1577x over previous
import functools

import jax
import jax.numpy as jnp
from jax import lax
from jax.experimental import pallas as pl
from jax.experimental.pallas import tpu as pltpu
from jax.experimental.pallas import tpu_sc as plsc

N_NODES = 10000
N_REL = 16
N_EDGES = 320000
N_BASIS = 8

NC = 2
NS = 16
NW = NC * NS

E_PER_W = N_EDGES // NW
BLK_E = 2000
N_BLK_E = E_PER_W // BLK_E
GRP = 80
N_GRP = BLK_E // GRP
ROWS_PER_TEC = N_NODES // NS
ZROWS = 125

LANE = 16


def _mesh():
    return plsc.VectorSubcoreMesh(core_axis_name="c", subcore_axis_name="s")


def _counts_inv(dst, etype):
    kern = pl.kernel(
        _counts_inv_body,
        out_type=jax.ShapeDtypeStruct((N_EDGES,), jnp.float32),
        mesh=_mesh(),
        scratch_types=dict(
            counts_sp=pltpu.VMEM_SHARED((N_REL * N_NODES,), jnp.float32),
            et_v=pltpu.VMEM((BLK_E,), jnp.int32),
            dst_v=pltpu.VMEM((BLK_E,), jnp.int32),
            key2d_v=pltpu.VMEM((N_GRP, GRP), jnp.int32),
            ones_v=pltpu.VMEM((GRP,), jnp.float32),
            cv_v=pltpu.VMEM((GRP,), jnp.float32),
            ibuf_v=pltpu.VMEM((BLK_E,), jnp.float32),
            zbuf_v=pltpu.VMEM((BLK_E,), jnp.float32),
            sem=pltpu.SemaphoreType.DMA,
        ),
        compiler_params=pltpu.CompilerParams(use_tc_tiling_on_sc=False),
    )
    return kern(dst, etype)


def _counts_inv_body(dst_hbm, et_hbm, inv_hbm,
                     counts_sp, et_v, dst_v, key2d_v, ones_v, cv_v,
                     ibuf_v, zbuf_v, sem):
    cid = lax.axis_index("c")
    sid = lax.axis_index("s")
    wid = sid * NC + cid

    z16 = jnp.zeros((LANE,), jnp.float32)
    o16 = jnp.ones((LANE,), jnp.float32)
    for g in range(GRP // LANE):
        ones_v[pl.ds(g * LANE, LANE)] = o16

    def zero_blk(i, _):
        zbuf_v[pl.ds(i * LANE, LANE)] = z16
        return 0

    lax.fori_loop(0, BLK_E // LANE, zero_blk, 0)
    for t in range(N_REL * N_NODES // NS // BLK_E):
        pltpu.sync_copy(zbuf_v,
                        counts_sp.at[pl.ds(sid * (N_REL * N_NODES // NS)
                                           + t * BLK_E, BLK_E)])
    plsc.subcore_barrier()

    def load_keys(base):
        pltpu.sync_copy(et_hbm.at[pl.ds(base, BLK_E)], et_v)
        pltpu.sync_copy(dst_hbm.at[pl.ds(base, BLK_E)], dst_v)
        for i in range(N_GRP):
            for j in range(GRP // LANE):
                off = i * GRP + j * LANE
                key = (et_v[pl.ds(off, LANE)] * N_NODES
                       + dst_v[pl.ds(off, LANE)])
                key2d_v[i, pl.ds(j * LANE, LANE)] = key

    def count_blk(b, _):
        base = sid * (N_EDGES // NS) + b * BLK_E
        load_keys(base)
        for i in range(N_GRP):
            pltpu.sync_copy(ones_v, counts_sp.at[key2d_v.at[i]], add=True)
        return 0

    lax.fori_loop(0, N_EDGES // NS // BLK_E, count_blk, 0)
    plsc.subcore_barrier()

    def inv_blk(b, _):
        base = wid * E_PER_W + b * BLK_E
        load_keys(base)
        for i in range(N_GRP):
            pltpu.async_copy(counts_sp.at[key2d_v.at[i]], cv_v, sem).wait()
            for j in range(GRP // LANE):
                c = cv_v[pl.ds(j * LANE, LANE)]
                ibuf_v[pl.ds(i * GRP + j * LANE, LANE)] = (
                    1.0 / jnp.maximum(c, 1.0))
        pltpu.sync_copy(ibuf_v, inv_hbm.at[pl.ds(base, BLK_E)])
        return 0

    lax.fori_loop(0, N_BLK_E, inv_blk, 0)


def _edge_pass(table, src, etype, inv, dst, d):
    kern = pl.kernel(
        functools.partial(_edge_pass_body, d=d),
        out_type=jax.ShapeDtypeStruct((NC, NS, ROWS_PER_TEC, d), jnp.float32),
        mesh=_mesh(),
        scratch_types=dict(
            agg_sp=pltpu.VMEM_SHARED((N_NODES, d), jnp.float32),
            src_v=pltpu.VMEM((BLK_E,), jnp.int32),
            et_v=pltpu.VMEM((BLK_E,), jnp.int32),
            dst_v=pltpu.VMEM((BLK_E,), jnp.int32),
            inv_v=pltpu.VMEM((BLK_E,), jnp.float32),
            dst2d_v=pltpu.VMEM((N_GRP, GRP), jnp.int32),
            key2d_v=pltpu.VMEM((N_GRP, GRP), jnp.int32),
            rows_v=pltpu.VMEM((GRP, d), jnp.float32),
            zbuf_v=pltpu.VMEM((ZROWS, d), jnp.float32),
            sem=pltpu.SemaphoreType.DMA,
        ),
        compiler_params=pltpu.CompilerParams(use_tc_tiling_on_sc=False),
    )
    return kern(table, src, etype, inv, dst).reshape(NC, N_NODES, d)


def _edge_pass_body(table_hbm, src_hbm, et_hbm, inv_hbm, dst_hbm, out_hbm,
                    agg_sp, src_v, et_v, dst_v, inv_v, dst2d_v, key2d_v,
                    rows_v, zbuf_v, sem, *, d):
    cid = lax.axis_index("c")
    sid = lax.axis_index("s")
    wid = sid * NC + cid
    nk = d // LANE

    z16 = jnp.zeros((LANE,), jnp.float32)

    def zero_row(i, _):
        for k in range(nk):
            zbuf_v[i, pl.ds(k * LANE, LANE)] = z16
        return 0

    lax.fori_loop(0, ZROWS, zero_row, 0)
    for t in range(ROWS_PER_TEC // ZROWS):
        pltpu.sync_copy(zbuf_v,
                        agg_sp.at[pl.ds(sid * ROWS_PER_TEC + t * ZROWS,
                                        ZROWS)])
    plsc.subcore_barrier()

    def blk(b, _):
        base = wid * E_PER_W + b * BLK_E
        pltpu.sync_copy(src_hbm.at[pl.ds(base, BLK_E)], src_v)
        pltpu.sync_copy(et_hbm.at[pl.ds(base, BLK_E)], et_v)
        pltpu.sync_copy(inv_hbm.at[pl.ds(base, BLK_E)], inv_v)
        pltpu.sync_copy(dst_hbm.at[pl.ds(base, BLK_E)], dst_v)
        for i in range(N_GRP):
            for j in range(GRP // LANE):
                off = i * GRP + j * LANE
                key = (et_v[pl.ds(off, LANE)] * N_NODES
                       + src_v[pl.ds(off, LANE)])
                key2d_v[i, pl.ds(j * LANE, LANE)] = key
                dst2d_v[i, pl.ds(j * LANE, LANE)] = dst_v[pl.ds(off, LANE)]
        for i in range(N_GRP):
            pltpu.async_copy(table_hbm.at[key2d_v.at[i]], rows_v, sem).wait()

            def scale(g, _):
                inv16 = inv_v[pl.ds(i * GRP + g * LANE, LANE)]
                for e in range(LANE):
                    s = inv16[e]
                    row = g * LANE + e
                    for k in range(nk):
                        rows_v[row, pl.ds(k * LANE, LANE)] = (
                            rows_v[row, pl.ds(k * LANE, LANE)] * s)
                return 0

            lax.fori_loop(0, GRP // LANE, scale, 0)
            pltpu.sync_copy(rows_v, agg_sp.at[dst2d_v.at[i]], add=True)
        return 0

    lax.fori_loop(0, N_BLK_E, blk, 0)
    plsc.subcore_barrier()
    pltpu.sync_copy(agg_sp.at[pl.ds(sid * ROWS_PER_TEC, ROWS_PER_TEC)],
                    out_hbm.at[cid, sid])


BLK_N = 2000


def _dense_rel(x, comp, bases):
    r, nb = comp.shape
    _, cin, cout = bases.shape
    n = x.shape[0]

    def body(comp_ref, bases_ref, x_ref, out_ref):
        i = pl.program_id(0)
        w = comp_ref[i, 0] * bases_ref[0]
        for b in range(1, nb):
            w = w + comp_ref[i, b] * bases_ref[b]
        out_ref[0] = jnp.dot(x_ref[...], w,
                             preferred_element_type=jnp.float32)

    return pl.pallas_call(
        body,
        grid=(r, n // BLK_N),
        in_specs=[
            pl.BlockSpec((r, nb), lambda i, j: (0, 0),
                         memory_space=pltpu.SMEM),
            pl.BlockSpec((nb, cin, cout), lambda i, j: (0, 0, 0)),
            pl.BlockSpec((BLK_N, cin), lambda i, j: (j, 0)),
        ],
        out_specs=pl.BlockSpec((1, BLK_N, cout), lambda i, j: (i, j, 0)),
        out_shape=jax.ShapeDtypeStruct((r, n, cout), jnp.float32),
    )(comp, bases, x)


def _dense_root(x, root, bias):
    cin, cout = root.shape
    n = x.shape[0]

    def body(root_ref, bias_ref, x_ref, out_ref):
        out_ref[...] = (jnp.dot(x_ref[...], root_ref[...],
                                preferred_element_type=jnp.float32)
                        + bias_ref[...])

    return pl.pallas_call(
        body,
        grid=(n // BLK_N,),
        in_specs=[
            pl.BlockSpec((cin, cout), lambda j: (0, 0)),
            pl.BlockSpec((1, cout), lambda j: (0, 0)),
            pl.BlockSpec((BLK_N, cin), lambda j: (j, 0)),
        ],
        out_specs=pl.BlockSpec((BLK_N, cout), lambda j: (j, 0)),
        out_shape=jax.ShapeDtypeStruct((n, cout), jnp.float32),
    )(root, bias.reshape(1, cout), x)


def _combine(parts, xr, relu):
    _, n, cout = parts.shape

    def body(parts_ref, xr_ref, out_ref):
        s = parts_ref[0] + parts_ref[1] + xr_ref[...]
        out_ref[...] = jnp.maximum(s, 0.0) if relu else s

    return pl.pallas_call(
        body,
        grid=(n // BLK_N,),
        in_specs=[
            pl.BlockSpec((NC, BLK_N, cout), lambda j: (0, j, 0)),
            pl.BlockSpec((BLK_N, cout), lambda j: (j, 0)),
        ],
        out_specs=pl.BlockSpec((BLK_N, cout), lambda j: (j, 0)),
        out_shape=jax.ShapeDtypeStruct((n, cout), jnp.float32),
    )(parts, xr)


def kernel(x, bases1, comp1, root1, bias1, bases2, comp2, root2, bias2,
           edge_index, edge_type):
    src = edge_index[0]
    dst = edge_index[1]

    inv = _counts_inv(dst, edge_type)

    xt1 = _dense_rel(x, comp1, bases1)
    xr1 = _dense_root(x, root1, bias1)
    p1 = _edge_pass(xt1.reshape(N_REL * N_NODES, xt1.shape[2]),
                    src, edge_type, inv, dst, xt1.shape[2])
    h = _combine(p1, xr1, relu=True)

    xt2 = _dense_rel(h, comp2, bases2)
    xr2 = _dense_root(h, root2, bias2)
    p2 = _edge_pass(xt2.reshape(N_REL * N_NODES, xt2.shape[2]),
                    src, edge_type, inv, dst, xt2.shape[2])
    return _combine(p2, xr2, relu=False)


# speedup vs baseline: 28.8890x; 1.2475x over previous
import functools

import jax
import jax.numpy as jnp
from jax import lax
from jax.experimental import pallas as pl
from jax.experimental.pallas import tpu as pltpu
from jax.experimental.pallas import tpu_sc as plsc

N_NODES = 10000
N_REL = 16
N_EDGES = 320000
N_BASIS = 8

NC = 2
NS = 16
NW = NC * NS

E_PER_W = N_EDGES // NW
BLK_E = 2000
N_BLK_E = E_PER_W // BLK_E
GRP = 80
N_GRP = BLK_E // GRP
ROWS_PER_TEC = N_NODES // NS
ZROWS = 125

LANE = 16


def _mesh():
    return plsc.VectorSubcoreMesh(core_axis_name="c", subcore_axis_name="s")


def _build_keys(et_v, a_v, key2d_v, mult):
    for i in range(N_GRP):
        for j in range(GRP // LANE):
            off = i * GRP + j * LANE
            key2d_v[i, pl.ds(j * LANE, LANE)] = (
                et_v[pl.ds(off, LANE)] * mult + a_v[pl.ds(off, LANE)])


def _counts_inv(dst, etype):
    kern = pl.kernel(
        _counts_inv_body,
        out_type=jax.ShapeDtypeStruct((N_EDGES,), jnp.float32),
        mesh=_mesh(),
        scratch_types=dict(
            counts_sp=pltpu.VMEM_SHARED((N_REL * N_NODES,), jnp.float32),
            et_v=pltpu.VMEM((BLK_E,), jnp.int32),
            dst_v=pltpu.VMEM((BLK_E,), jnp.int32),
            key2d_v=pltpu.VMEM((N_GRP, GRP), jnp.int32),
            ones_v=pltpu.VMEM((GRP,), jnp.float32),
            cv0=pltpu.VMEM((GRP,), jnp.float32),
            cv1=pltpu.VMEM((GRP,), jnp.float32),
            ibuf_v=pltpu.VMEM((BLK_E,), jnp.float32),
            zbuf_v=pltpu.VMEM((BLK_E,), jnp.float32),
            sem_s=pltpu.SemaphoreType.DMA,
            sem_g0=pltpu.SemaphoreType.DMA,
            sem_g1=pltpu.SemaphoreType.DMA,
        ),
        compiler_params=pltpu.CompilerParams(use_tc_tiling_on_sc=False),
    )
    return kern(dst, etype)


def _counts_inv_body(dst_hbm, et_hbm, inv_hbm,
                     counts_sp, et_v, dst_v, key2d_v, ones_v, cv0, cv1,
                     ibuf_v, zbuf_v, sem_s, sem_g0, sem_g1):
    cid = lax.axis_index("c")
    sid = lax.axis_index("s")
    wid = sid * NC + cid

    z16 = jnp.zeros((LANE,), jnp.float32)
    o16 = jnp.ones((LANE,), jnp.float32)
    for g in range(GRP // LANE):
        ones_v[pl.ds(g * LANE, LANE)] = o16

    def zero_blk(i, _):
        zbuf_v[pl.ds(i * LANE, LANE)] = z16
        return 0

    lax.fori_loop(0, BLK_E // LANE, zero_blk, 0)
    for t in range(N_REL * N_NODES // NS // BLK_E):
        pltpu.sync_copy(zbuf_v,
                        counts_sp.at[pl.ds(sid * (N_REL * N_NODES // NS)
                                           + t * BLK_E, BLK_E)])
    plsc.subcore_barrier()

    def count_blk(b, _):
        base = sid * (N_EDGES // NS) + b * BLK_E
        pltpu.sync_copy(et_hbm.at[pl.ds(base, BLK_E)], et_v)
        pltpu.sync_copy(dst_hbm.at[pl.ds(base, BLK_E)], dst_v)
        _build_keys(et_v, dst_v, key2d_v, N_NODES)
        descs = [pltpu.async_copy(ones_v, counts_sp.at[key2d_v.at[i]],
                                  sem_s, add=True)
                 for i in range(N_GRP)]
        for d in descs:
            d.wait()
        return 0

    lax.fori_loop(0, N_EDGES // NS // BLK_E, count_blk, 0)
    plsc.subcore_barrier()

    def inv_blk(b, _):
        base = wid * E_PER_W + b * BLK_E
        pltpu.sync_copy(et_hbm.at[pl.ds(base, BLK_E)], et_v)
        pltpu.sync_copy(dst_hbm.at[pl.ds(base, BLK_E)], dst_v)
        _build_keys(et_v, dst_v, key2d_v, N_NODES)
        cv = [cv0, cv1]
        sems = [sem_g0, sem_g1]
        dg = [pltpu.async_copy(counts_sp.at[key2d_v.at[i]], cv[i], sems[i])
              for i in range(2)]
        for i in range(N_GRP):
            p = i % 2
            dg[p].wait()
            for j in range(GRP // LANE):
                c = cv[p][pl.ds(j * LANE, LANE)]
                ibuf_v[pl.ds(i * GRP + j * LANE, LANE)] = (
                    1.0 / jnp.maximum(c, 1.0))
            if i + 2 < N_GRP:
                dg[p] = pltpu.async_copy(counts_sp.at[key2d_v.at[i + 2]],
                                         cv[p], sems[p])
        pltpu.sync_copy(ibuf_v, inv_hbm.at[pl.ds(base, BLK_E)])
        return 0

    lax.fori_loop(0, N_BLK_E, inv_blk, 0)


def _edge_pass(table, src, etype, inv, dst, d):
    kern = pl.kernel(
        functools.partial(_edge_pass_body, d=d),
        out_type=jax.ShapeDtypeStruct((NC, NS, ROWS_PER_TEC, d), jnp.float32),
        mesh=_mesh(),
        scratch_types=dict(
            agg_sp=pltpu.VMEM_SHARED((N_NODES, d), jnp.float32),
            src_v=pltpu.VMEM((BLK_E,), jnp.int32),
            et_v=pltpu.VMEM((BLK_E,), jnp.int32),
            dst_v=pltpu.VMEM((BLK_E,), jnp.int32),
            inv_v=pltpu.VMEM((BLK_E,), jnp.float32),
            dst2d_v=pltpu.VMEM((N_GRP, GRP), jnp.int32),
            key2d_v=pltpu.VMEM((N_GRP, GRP), jnp.int32),
            rows0=pltpu.VMEM((GRP, d), jnp.float32),
            rows1=pltpu.VMEM((GRP, d), jnp.float32),
            rows2=pltpu.VMEM((GRP, d), jnp.float32),
            rows3=pltpu.VMEM((GRP, d), jnp.float32),
            zbuf_v=pltpu.VMEM((ZROWS, d), jnp.float32),
            sem_g0=pltpu.SemaphoreType.DMA,
            sem_g1=pltpu.SemaphoreType.DMA,
            sem_g2=pltpu.SemaphoreType.DMA,
            sem_g3=pltpu.SemaphoreType.DMA,
            sem_s0=pltpu.SemaphoreType.DMA,
            sem_s1=pltpu.SemaphoreType.DMA,
            sem_s2=pltpu.SemaphoreType.DMA,
            sem_s3=pltpu.SemaphoreType.DMA,
        ),
        compiler_params=pltpu.CompilerParams(use_tc_tiling_on_sc=False),
    )
    return kern(table, src, etype, inv, dst).reshape(NC, N_NODES, d)


def _edge_pass_body(table_hbm, src_hbm, et_hbm, inv_hbm, dst_hbm, out_hbm,
                    agg_sp, src_v, et_v, dst_v, inv_v, dst2d_v, key2d_v,
                    rows0, rows1, rows2, rows3, zbuf_v,
                    sem_g0, sem_g1, sem_g2, sem_g3,
                    sem_s0, sem_s1, sem_s2, sem_s3, *, d):
    cid = lax.axis_index("c")
    sid = lax.axis_index("s")
    wid = sid * NC + cid
    nk = d // LANE
    rows = [rows0, rows1, rows2, rows3]
    gsems = [sem_g0, sem_g1, sem_g2, sem_g3]
    ssems = [sem_s0, sem_s1, sem_s2, sem_s3]

    z16 = jnp.zeros((LANE,), jnp.float32)

    def zero_row(i, _):
        for k in range(nk):
            zbuf_v[i, pl.ds(k * LANE, LANE)] = z16
        return 0

    lax.fori_loop(0, ZROWS, zero_row, 0)
    for t in range(ROWS_PER_TEC // ZROWS):
        pltpu.sync_copy(zbuf_v,
                        agg_sp.at[pl.ds(sid * ROWS_PER_TEC + t * ZROWS,
                                        ZROWS)])
    plsc.subcore_barrier()

    def scale(buf, i):
        def body(g, _):
            inv16 = inv_v[pl.ds(i * GRP + g * LANE, LANE)]
            for e in range(LANE):
                s = inv16[e]
                row = g * LANE + e
                for k in range(nk):
                    buf[row, pl.ds(k * LANE, LANE)] = (
                        buf[row, pl.ds(k * LANE, LANE)] * s)
            return 0

        lax.fori_loop(0, GRP // LANE, body, 0)

    def blk(b, _):
        base = wid * E_PER_W + b * BLK_E
        pltpu.sync_copy(src_hbm.at[pl.ds(base, BLK_E)], src_v)
        pltpu.sync_copy(et_hbm.at[pl.ds(base, BLK_E)], et_v)
        pltpu.sync_copy(inv_hbm.at[pl.ds(base, BLK_E)], inv_v)
        pltpu.sync_copy(dst_hbm.at[pl.ds(base, BLK_E)], dst_v)
        _build_keys(et_v, src_v, key2d_v, N_NODES)
        for i in range(N_GRP):
            for j in range(GRP // LANE):
                off = i * GRP + j * LANE
                dst2d_v[i, pl.ds(j * LANE, LANE)] = dst_v[pl.ds(off, LANE)]

        def gather(i, p):
            return pltpu.async_copy(table_hbm.at[key2d_v.at[i]], rows[p],
                                    gsems[p])

        dg = [gather(0, 0), gather(1, 1), None, None]
        ds = [None, None, None, None]
        for i in range(N_GRP):
            p = i % 4
            dg[p].wait()
            scale(rows[p], i)
            ds[p] = pltpu.async_copy(rows[p], agg_sp.at[dst2d_v.at[i]],
                                     ssems[p], add=True)
            if i + 2 < N_GRP:
                q = (i + 2) % 4
                if ds[q] is not None:
                    ds[q].wait()
                    ds[q] = None
                dg[q] = gather(i + 2, q)
        for p in range(4):
            if ds[p] is not None:
                ds[p].wait()
        return 0

    lax.fori_loop(0, N_BLK_E, blk, 0)
    plsc.subcore_barrier()
    pltpu.sync_copy(agg_sp.at[pl.ds(sid * ROWS_PER_TEC, ROWS_PER_TEC)],
                    out_hbm.at[cid, sid])


BLK_N = 2000


def _dense(x, comp_aug, bases_aug):
    r, nb = comp_aug.shape
    _, cin, cout = bases_aug.shape
    n = x.shape[0]

    def body(comp_ref, bases_ref, x_ref, out_ref):
        i = pl.program_id(0)
        w = comp_ref[i, 0] * bases_ref[0]
        for b in range(1, nb):
            w = w + comp_ref[i, b] * bases_ref[b]
        out_ref[0] = jnp.dot(x_ref[...], w,
                             preferred_element_type=jnp.float32)

    return pl.pallas_call(
        body,
        grid=(r, n // BLK_N),
        in_specs=[
            pl.BlockSpec((r, nb), lambda i, j: (0, 0),
                         memory_space=pltpu.SMEM),
            pl.BlockSpec((nb, cin, cout), lambda i, j: (0, 0, 0)),
            pl.BlockSpec((BLK_N, cin), lambda i, j: (j, 0)),
        ],
        out_specs=pl.BlockSpec((1, BLK_N, cout), lambda i, j: (i, j, 0)),
        out_shape=jax.ShapeDtypeStruct((r, n, cout), jnp.float32),
    )(comp_aug, bases_aug, x)


def _combine(parts, xt_all, bias, relu):
    _, n, cout = parts.shape

    def body(parts_ref, xr_ref, bias_ref, out_ref):
        s = parts_ref[0] + parts_ref[1] + xr_ref[0] + bias_ref[...]
        out_ref[...] = jnp.maximum(s, 0.0) if relu else s

    return pl.pallas_call(
        body,
        grid=(n // BLK_N,),
        in_specs=[
            pl.BlockSpec((NC, BLK_N, cout), lambda j: (0, j, 0)),
            pl.BlockSpec((1, BLK_N, cout), lambda j: (N_REL, j, 0)),
            pl.BlockSpec((1, cout), lambda j: (0, 0)),
        ],
        out_specs=pl.BlockSpec((BLK_N, cout), lambda j: (j, 0)),
        out_shape=jax.ShapeDtypeStruct((n, cout), jnp.float32),
    )(parts, xt_all, bias.reshape(1, cout))


def _augment(comp, bases, root):
    nb = comp.shape[1]
    bases_aug = jnp.concatenate([bases, root[None]], axis=0)
    comp_aug = jnp.concatenate(
        [jnp.concatenate([comp, jnp.zeros((comp.shape[0], 1), comp.dtype)],
                         axis=1),
         jnp.zeros((1, nb + 1), comp.dtype).at[0, nb].set(1.0)],
        axis=0)
    return comp_aug, bases_aug


def kernel(x, bases1, comp1, root1, bias1, bases2, comp2, root2, bias2,
           edge_index, edge_type):
    src = edge_index[0]
    dst = edge_index[1]

    inv = _counts_inv(dst, edge_type)

    ca1, ba1 = _augment(comp1, bases1, root1)
    xt1 = _dense(x, ca1, ba1)
    d1 = xt1.shape[2]
    p1 = _edge_pass(xt1.reshape((N_REL + 1) * N_NODES, d1),
                    src, edge_type, inv, dst, d1)
    h = _combine(p1, xt1, bias1, relu=True)

    ca2, ba2 = _augment(comp2, bases2, root2)
    xt2 = _dense(h, ca2, ba2)
    d2 = xt2.shape[2]
    p2 = _edge_pass(xt2.reshape((N_REL + 1) * N_NODES, d2),
                    src, edge_type, inv, dst, d2)
    return _combine(p2, xt2, bias2, relu=False)


# speedup vs baseline: 45.5108x; 1.5754x over previous
import functools

import jax
import jax.numpy as jnp
from jax import lax
from jax.experimental import pallas as pl
from jax.experimental.pallas import tpu as pltpu
from jax.experimental.pallas import tpu_sc as plsc

N_NODES = 10000
N_REL = 16
N_EDGES = 320000
N_BASIS = 8

NC = 2
NS = 16
NW = NC * NS

E_PER_W = N_EDGES // NW
BLK_E = 2000
N_BLK_E = E_PER_W // BLK_E
GRP = 80
N_GRP = BLK_E // GRP
ROWS_PER_TEC = N_NODES // NS
ZROWS = 125

LANE = 16


def _mesh():
    return plsc.VectorSubcoreMesh(core_axis_name="c", subcore_axis_name="s")


def _build_keys(hi_v, lo_v, key2d_v, mult):
    for i in range(N_GRP):
        for j in range(GRP // LANE):
            off = i * GRP + j * LANE
            key2d_v[i, pl.ds(j * LANE, LANE)] = (
                hi_v[pl.ds(off, LANE)] * mult + lo_v[pl.ds(off, LANE)])


def _counts_inv(dst, etype):
    kern = pl.kernel(
        _counts_inv_body,
        out_type=jax.ShapeDtypeStruct((N_EDGES,), jnp.float32),
        mesh=_mesh(),
        scratch_types=dict(
            counts_sp=pltpu.VMEM_SHARED((N_REL * N_NODES,), jnp.float32),
            et_v=pltpu.VMEM((BLK_E,), jnp.int32),
            dst_v=pltpu.VMEM((BLK_E,), jnp.int32),
            key2d_v=pltpu.VMEM((N_GRP, GRP), jnp.int32),
            ones_v=pltpu.VMEM((GRP,), jnp.float32),
            cv0=pltpu.VMEM((GRP,), jnp.float32),
            cv1=pltpu.VMEM((GRP,), jnp.float32),
            ibuf_v=pltpu.VMEM((BLK_E,), jnp.float32),
            zbuf_v=pltpu.VMEM((BLK_E,), jnp.float32),
            sem_s=pltpu.SemaphoreType.DMA,
            sem_g0=pltpu.SemaphoreType.DMA,
            sem_g1=pltpu.SemaphoreType.DMA,
        ),
        compiler_params=pltpu.CompilerParams(use_tc_tiling_on_sc=False),
    )
    return kern(dst, etype)


def _counts_inv_body(dst_hbm, et_hbm, inv_hbm,
                     counts_sp, et_v, dst_v, key2d_v, ones_v, cv0, cv1,
                     ibuf_v, zbuf_v, sem_s, sem_g0, sem_g1):
    cid = lax.axis_index("c")
    sid = lax.axis_index("s")
    wid = sid * NC + cid

    z16 = jnp.zeros((LANE,), jnp.float32)
    o16 = jnp.ones((LANE,), jnp.float32)
    for g in range(GRP // LANE):
        ones_v[pl.ds(g * LANE, LANE)] = o16

    def zero_blk(i, _):
        zbuf_v[pl.ds(i * LANE, LANE)] = z16
        return 0

    lax.fori_loop(0, BLK_E // LANE, zero_blk, 0)
    for t in range(N_REL * N_NODES // NS // BLK_E):
        pltpu.sync_copy(zbuf_v,
                        counts_sp.at[pl.ds(sid * (N_REL * N_NODES // NS)
                                           + t * BLK_E, BLK_E)])
    plsc.subcore_barrier()

    def count_blk(b, _):
        base = sid * (N_EDGES // NS) + b * BLK_E
        pltpu.sync_copy(et_hbm.at[pl.ds(base, BLK_E)], et_v)
        pltpu.sync_copy(dst_hbm.at[pl.ds(base, BLK_E)], dst_v)
        _build_keys(et_v, dst_v, key2d_v, N_NODES)
        descs = [pltpu.async_copy(ones_v, counts_sp.at[key2d_v.at[i]],
                                  sem_s, add=True)
                 for i in range(N_GRP)]
        for d in descs:
            d.wait()
        return 0

    lax.fori_loop(0, N_EDGES // NS // BLK_E, count_blk, 0)
    plsc.subcore_barrier()

    def inv_blk(b, _):
        base = wid * E_PER_W + b * BLK_E
        pltpu.sync_copy(et_hbm.at[pl.ds(base, BLK_E)], et_v)
        pltpu.sync_copy(dst_hbm.at[pl.ds(base, BLK_E)], dst_v)
        _build_keys(et_v, dst_v, key2d_v, N_NODES)
        cv = [cv0, cv1]
        sems = [sem_g0, sem_g1]
        dg = [pltpu.async_copy(counts_sp.at[key2d_v.at[i]], cv[i], sems[i])
              for i in range(2)]
        for i in range(N_GRP):
            p = i % 2
            dg[p].wait()
            for j in range(GRP // LANE):
                c = cv[p][pl.ds(j * LANE, LANE)]
                ibuf_v[pl.ds(i * GRP + j * LANE, LANE)] = (
                    1.0 / jnp.maximum(c, 1.0))
            if i + 2 < N_GRP:
                dg[p] = pltpu.async_copy(counts_sp.at[key2d_v.at[i + 2]],
                                         cv[p], sems[p])
        pltpu.sync_copy(ibuf_v, inv_hbm.at[pl.ds(base, BLK_E)])
        return 0

    lax.fori_loop(0, N_BLK_E, inv_blk, 0)


def _edge_pass(table, src, etype, inv, dst, d):
    kern = pl.kernel(
        functools.partial(_edge_pass_body, d=d),
        out_type=jax.ShapeDtypeStruct((NC, NS, ROWS_PER_TEC, d), jnp.float32),
        mesh=_mesh(),
        scratch_types=dict(
            agg_sp=pltpu.VMEM_SHARED((N_NODES, d), jnp.float32),
            src_v=pltpu.VMEM((BLK_E,), jnp.int32),
            et_v=pltpu.VMEM((BLK_E,), jnp.int32),
            dst_v=pltpu.VMEM((BLK_E,), jnp.int32),
            inv_v=pltpu.VMEM((BLK_E,), jnp.float32),
            dst2d_v=pltpu.VMEM((N_GRP, GRP), jnp.int32),
            key2d_v=pltpu.VMEM((N_GRP, GRP), jnp.int32),
            rows0=pltpu.VMEM((GRP, d), jnp.float32),
            rows1=pltpu.VMEM((GRP, d), jnp.float32),
            rows2=pltpu.VMEM((GRP, d), jnp.float32),
            rows3=pltpu.VMEM((GRP, d), jnp.float32),
            zbuf_v=pltpu.VMEM((ZROWS, d), jnp.float32),
            sem_g0=pltpu.SemaphoreType.DMA,
            sem_g1=pltpu.SemaphoreType.DMA,
            sem_g2=pltpu.SemaphoreType.DMA,
            sem_g3=pltpu.SemaphoreType.DMA,
            sem_s0=pltpu.SemaphoreType.DMA,
            sem_s1=pltpu.SemaphoreType.DMA,
            sem_s2=pltpu.SemaphoreType.DMA,
            sem_s3=pltpu.SemaphoreType.DMA,
        ),
        compiler_params=pltpu.CompilerParams(use_tc_tiling_on_sc=False),
    )
    return kern(table, src, etype, inv, dst).reshape(NC, N_NODES, d)


def _edge_pass_body(table_hbm, src_hbm, et_hbm, inv_hbm, dst_hbm, out_hbm,
                    agg_sp, src_v, et_v, dst_v, inv_v, dst2d_v, key2d_v,
                    rows0, rows1, rows2, rows3, zbuf_v,
                    sem_g0, sem_g1, sem_g2, sem_g3,
                    sem_s0, sem_s1, sem_s2, sem_s3, *, d):
    cid = lax.axis_index("c")
    sid = lax.axis_index("s")
    wid = sid * NC + cid
    nk = d // LANE
    rows = [rows0, rows1, rows2, rows3]
    gsems = [sem_g0, sem_g1, sem_g2, sem_g3]
    ssems = [sem_s0, sem_s1, sem_s2, sem_s3]

    z16 = jnp.zeros((LANE,), jnp.float32)

    def zero_row(i, _):
        for k in range(nk):
            zbuf_v[i, pl.ds(k * LANE, LANE)] = z16
        return 0

    lax.fori_loop(0, ZROWS, zero_row, 0)
    for t in range(ROWS_PER_TEC // ZROWS):
        pltpu.sync_copy(zbuf_v,
                        agg_sp.at[pl.ds(sid * ROWS_PER_TEC + t * ZROWS,
                                        ZROWS)])
    plsc.subcore_barrier()

    def scale(buf, i):
        def body(g, _):
            inv16 = inv_v[pl.ds(i * GRP + g * LANE, LANE)]
            for e in range(LANE):
                s = inv16[e]
                row = g * LANE + e
                for k in range(nk):
                    buf[row, pl.ds(k * LANE, LANE)] = (
                        buf[row, pl.ds(k * LANE, LANE)] * s)
            return 0

        lax.fori_loop(0, GRP // LANE, body, 0)

    def blk(b, _):
        base = wid * E_PER_W + b * BLK_E
        pltpu.sync_copy(src_hbm.at[pl.ds(base, BLK_E)], src_v)
        pltpu.sync_copy(et_hbm.at[pl.ds(base, BLK_E)], et_v)
        pltpu.sync_copy(inv_hbm.at[pl.ds(base, BLK_E)], inv_v)
        pltpu.sync_copy(dst_hbm.at[pl.ds(base, BLK_E)], dst_v)
        _build_keys(src_v, et_v, key2d_v, N_REL + 1)
        for i in range(N_GRP):
            for j in range(GRP // LANE):
                off = i * GRP + j * LANE
                dst2d_v[i, pl.ds(j * LANE, LANE)] = dst_v[pl.ds(off, LANE)]

        def gather(i, p):
            return pltpu.async_copy(table_hbm.at[key2d_v.at[i]], rows[p],
                                    gsems[p])

        dg = [gather(0, 0), gather(1, 1), None, None]
        ds = [None, None, None, None]
        for i in range(N_GRP):
            p = i % 4
            dg[p].wait()
            scale(rows[p], i)
            ds[p] = pltpu.async_copy(rows[p], agg_sp.at[dst2d_v.at[i]],
                                     ssems[p], add=True)
            if i + 2 < N_GRP:
                q = (i + 2) % 4
                if ds[q] is not None:
                    ds[q].wait()
                    ds[q] = None
                dg[q] = gather(i + 2, q)
        for p in range(4):
            if ds[p] is not None:
                ds[p].wait()
        return 0

    lax.fori_loop(0, N_BLK_E, blk, 0)
    plsc.subcore_barrier()
    pltpu.sync_copy(agg_sp.at[pl.ds(sid * ROWS_PER_TEC, ROWS_PER_TEC)],
                    out_hbm.at[cid, sid])


BLK_N = 2000


def _dense(x, comp_aug, bases_aug):
    r, nb = comp_aug.shape
    _, cin, cout = bases_aug.shape
    n = x.shape[0]

    def body(comp_ref, bases_ref, x_ref, out_ref, wcat_ref):
        @pl.when(pl.program_id(0) == 0)
        def _build():
            for rr in range(r):
                w = comp_ref[rr, 0] * bases_ref[0]
                for b in range(1, nb):
                    w = w + comp_ref[rr, b] * bases_ref[b]
                wcat_ref[:, rr * cout:(rr + 1) * cout] = w

        out_ref[...] = jnp.dot(x_ref[...], wcat_ref[...],
                               preferred_element_type=jnp.float32)

    return pl.pallas_call(
        body,
        grid=(n // BLK_N,),
        in_specs=[
            pl.BlockSpec((r, nb), lambda j: (0, 0),
                         memory_space=pltpu.SMEM),
            pl.BlockSpec((nb, cin, cout), lambda j: (0, 0, 0)),
            pl.BlockSpec((BLK_N, cin), lambda j: (j, 0)),
        ],
        out_specs=pl.BlockSpec((BLK_N, r * cout), lambda j: (j, 0)),
        out_shape=jax.ShapeDtypeStruct((n, r * cout), jnp.float32),
        scratch_shapes=[pltpu.VMEM((cin, r * cout), jnp.float32)],
    )(comp_aug, bases_aug, x)


def _combine(parts, xt_all, bias, relu):
    _, n, cout = parts.shape

    def body(parts_ref, xr_ref, bias_ref, out_ref):
        s = (parts_ref[0] + parts_ref[1] + xr_ref[:, :cout]
             + bias_ref[...])
        out_ref[...] = jnp.maximum(s, 0.0) if relu else s

    return pl.pallas_call(
        body,
        grid=(n // BLK_N,),
        in_specs=[
            pl.BlockSpec((NC, BLK_N, cout), lambda j: (0, j, 0)),
            pl.BlockSpec((BLK_N, 128), lambda j: (j, (N_REL * cout) // 128)),
            pl.BlockSpec((1, cout), lambda j: (0, 0)),
        ],
        out_specs=pl.BlockSpec((BLK_N, cout), lambda j: (j, 0)),
        out_shape=jax.ShapeDtypeStruct((n, cout), jnp.float32),
    )(parts, xt_all, bias.reshape(1, cout))


def _augment(comp, bases, root):
    nb = comp.shape[1]
    bases_aug = jnp.concatenate([bases, root[None]], axis=0)
    comp_aug = jnp.concatenate(
        [jnp.concatenate([comp, jnp.zeros((comp.shape[0], 1), comp.dtype)],
                         axis=1),
         jnp.zeros((1, nb + 1), comp.dtype).at[0, nb].set(1.0)],
        axis=0)
    return comp_aug, bases_aug


def kernel(x, bases1, comp1, root1, bias1, bases2, comp2, root2, bias2,
           edge_index, edge_type):
    src = edge_index[0]
    dst = edge_index[1]

    inv = _counts_inv(dst, edge_type)

    ca1, ba1 = _augment(comp1, bases1, root1)
    xt1 = _dense(x, ca1, ba1)
    d1 = bases1.shape[2]
    p1 = _edge_pass(xt1.reshape(N_NODES * (N_REL + 1), d1),
                    src, edge_type, inv, dst, d1)
    h = _combine(p1, xt1, bias1, relu=True)

    ca2, ba2 = _augment(comp2, bases2, root2)
    xt2 = _dense(h, ca2, ba2)
    d2 = bases2.shape[2]
    p2 = _edge_pass(xt2.reshape(N_NODES * (N_REL + 1), d2),
                    src, edge_type, inv, dst, d2)
    return _combine(p2, xt2, bias2, relu=False)
